# baseline (speedup 1.0000x reference)
import functools

import jax
import jax.numpy as jnp
from jax import lax
from jax.experimental import pallas as pl
from jax.experimental.pallas import tpu as pltpu
from jax.experimental.pallas import tpu_sc as plsc

N = 10000
E = 320000
D = 128
EPS = 1e-12

NC = 2
NS = 16
L = 16
NW = NC * NS

CHA = 16
EPW_A = E // NW
NCH_A = EPW_A // CHA
CHB = 40
EPS_B = E // NS
NCH_B = EPS_B // CHB
SLAB = 624
ZR = 16
NZ = SLAB // ZR
TAIL = N - NS * SLAB

_mesh = plsc.VectorSubcoreMesh(core_axis_name="c", subcore_axis_name="s")


def _sigmoid(x):
    return 1.0 / (1.0 + jnp.exp(-x))


def _lanesum(v):
    lanes = lax.iota(jnp.int32, L)
    for sh in (1, 2, 4, 8):
        v = v + jnp.take(v, lanes ^ sh, axis=0)
    return v


def _node_dense_body(h_ref, p_ref, VW1, VW2, Vb, KW1, KW2, Kb, B1W, B1b, B2W,
                     B2b, C1W, C1b, C2W, C2b, vh_o, c1_o, c2_o, srcA_o,
                     dstA_o):
    h = h_ref[...]
    p = p_ref[...]
    vh_o[...] = h @ VW1[...] + p @ VW2[...] + Vb[...]
    qh = h @ KW1[...] + p @ KW2[...] + Kb[...]
    srcA_o[:, :D] = h @ B1W[...] + B1b[...]
    srcA_o[:, D:] = jnp.exp(jnp.tanh(qh))
    dstA_o[:, :D] = h @ B2W[...] + B2b[...]
    dstA_o[:, D:] = jnp.exp(_sigmoid(qh))
    c1_o[...] = p @ C1W[...] + C1b[...]
    c2_o[...] = p @ C2W[...] + C2b[...]


def _node_dense(h, p, VW, Vb, KW, Kb, B1W, B1b, B2W, B2b, C1W, C1b, C2W, C2b):
    R = 2000
    grid = N // R
    row = pl.BlockSpec((R, D), lambda i: (i, 0))
    row2 = pl.BlockSpec((R, 2 * D), lambda i: (i, 0))
    full = pl.BlockSpec((D, D), lambda i: (0, 0))
    vec = pl.BlockSpec((D,), lambda i: (0,))
    return pl.pallas_call(
        _node_dense_body,
        grid=grid,
        in_specs=[row, row, full, full, vec, full, full, vec,
                  full, vec, full, vec, full, vec, full, vec],
        out_specs=[row, row, row, row2, row2],
        out_shape=[jax.ShapeDtypeStruct((N, D), jnp.float32),
                   jax.ShapeDtypeStruct((N, D), jnp.float32),
                   jax.ShapeDtypeStruct((N, D), jnp.float32),
                   jax.ShapeDtypeStruct((N, 2 * D), jnp.float32),
                   jax.ShapeDtypeStruct((N, 2 * D), jnp.float32)],
    )(h, p, VW[:D], VW[D:], Vb, KW[:D], KW[D:], Kb, B1W, B1b, B2W, B2b,
      C1W, C1b, C2W, C2b)


def _edge_dense_body(e_ref, W, b, o_ref):
    o_ref[...] = e_ref[...] @ W[...] + b[...]


def _edge_dense(e, B3W, B3b):
    R = 2000
    return pl.pallas_call(
        _edge_dense_body,
        grid=E // R,
        in_specs=[pl.BlockSpec((R, D), lambda i: (i, 0)),
                  pl.BlockSpec((D, D), lambda i: (0, 0)),
                  pl.BlockSpec((D,), lambda i: (0,))],
        out_specs=pl.BlockSpec((R, D), lambda i: (i, 0)),
        out_shape=jax.ShapeDtypeStruct((E, D), jnp.float32),
    )(e, B3W, B3b)


def _estats_body(he_ref, st_o):
    i = pl.program_id(0)

    @pl.when(i == 0)
    def _():
        st_o[...] = jnp.zeros_like(st_o)

    he = he_ref[...]
    st_o[0, :] += jnp.sum(he, axis=0)
    st_o[1, :] += jnp.sum(he * he, axis=0)


def _estats(hat_eta):
    R = 4000
    return pl.pallas_call(
        _estats_body,
        grid=E // R,
        in_specs=[pl.BlockSpec((R, D), lambda i: (i, 0))],
        out_specs=pl.BlockSpec((2, D), lambda i: (0, 0)),
        out_shape=jax.ShapeDtypeStruct((2, D), jnp.float32),
    )(hat_eta)


def _finalize_body(vh_ref, c1_ref, acch_ref, accp_ref, ssig_ref, st_ref,
                   bnh_g, bnh_b, ln_g, ln_b, h_o, p_o, em_o, ei_o):
    den = ssig_ref[0] + ssig_ref[1] + EPS
    h = vh_ref[...] + acch_ref[...] / den
    m = jnp.mean(h, axis=0, keepdims=True)
    v = jnp.mean((h - m) ** 2, axis=0, keepdims=True)
    h = (h - m) * lax.rsqrt(v + 1e-5) * bnh_g[...] + bnh_b[...]
    h = jnp.maximum(h, 0.0)
    lm = jnp.mean(h, axis=-1, keepdims=True)
    lv = jnp.mean((h - lm) ** 2, axis=-1, keepdims=True)
    h_o[...] = (h - lm) * lax.rsqrt(lv + 1e-5) * ln_g[...] + ln_b[...]
    p_o[...] = jnp.tanh(c1_ref[...] + accp_ref[...] / den)
    em = st_ref[0] / E
    ev = st_ref[1] / E - em * em
    em_o[...] = em.reshape(1, D)
    ei_o[...] = lax.rsqrt(ev + 1e-5).reshape(1, D)


def _finalize(vh, c1p, acc_h, acc_p, ssig_parts, stats, bnh_g, bnh_b, ln_g,
              ln_b):
    nodes = pl.BlockSpec((N, D), lambda: (0, 0))
    vec = pl.BlockSpec((D,), lambda: (0,))
    return pl.pallas_call(
        _finalize_body,
        in_specs=[nodes, nodes, nodes, nodes,
                  pl.BlockSpec((NC, N, D), lambda: (0, 0, 0)),
                  pl.BlockSpec((2, D), lambda: (0, 0)),
                  vec, vec, vec, vec],
        out_specs=[nodes, nodes, pl.BlockSpec((1, D), lambda: (0, 0)),
                   pl.BlockSpec((1, D), lambda: (0, 0))],
        out_shape=[jax.ShapeDtypeStruct((N, D), jnp.float32),
                   jax.ShapeDtypeStruct((N, D), jnp.float32),
                   jax.ShapeDtypeStruct((1, D), jnp.float32),
                   jax.ShapeDtypeStruct((1, D), jnp.float32)],
    )(vh, c1p, acc_h, acc_p, ssig_parts, stats, bnh_g, bnh_b, ln_g, ln_b)


def _enew_body(he_ref, em_ref, ei_ref, g, b, o_ref):
    x = (he_ref[...] - em_ref[...]) * ei_ref[...] * g[...] + b[...]
    o_ref[...] = jnp.maximum(x, 0.0)


def _enew(hat_eta, em, ei, bne_g, bne_b):
    R = 2000
    return pl.pallas_call(
        _enew_body,
        grid=E // R,
        in_specs=[pl.BlockSpec((R, D), lambda i: (i, 0)),
                  pl.BlockSpec((1, D), lambda i: (0, 0)),
                  pl.BlockSpec((1, D), lambda i: (0, 0)),
                  pl.BlockSpec((D,), lambda i: (0,)),
                  pl.BlockSpec((D,), lambda i: (0,))],
        out_specs=pl.BlockSpec((R, D), lambda i: (i, 0)),
        out_shape=jax.ShapeDtypeStruct((E, D), jnp.float32),
    )(hat_eta, em, ei, bne_g, bne_b)


def _pass_a_body(b3e_h, src_h, dst_h, srcA_h, dstA_h,
                 he_h, al_h, ssig_h,
                 is0, id0, src0, dst0, eb0, sg0, ab0,
                 is1, id1, src1, dst1, eb1, sg1, ab1,
                 zbuf, acc,
                 m_gs0, m_gd0, m_e0, m_sc0, m_wh0, m_wa0,
                 m_gs1, m_gd1, m_e1, m_sc1, m_wh1, m_wa1,
                 m_i0, m_i1, m_i2, m_i3):
    c = lax.axis_index("c")
    s = lax.axis_index("s")
    wid = s * NC + c
    base = wid * EPW_A

    zv = jnp.zeros((L,), jnp.float32)

    def zb(i, _):
        r = i // 8
        col = (i % 8) * L
        zbuf[r, pl.ds(col, L)] = zv
        return 0

    lax.fori_loop(0, ZR * 8, zb, 0)
    for k in range(NZ):
        pltpu.sync_copy(zbuf, acc.at[pl.ds(s * SLAB + k * ZR, ZR)])

    @pl.when(s == 0)
    def _():
        pltpu.sync_copy(zbuf.at[pl.ds(0, TAIL)], acc.at[pl.ds(NS * SLAB, TAIL)])

    plsc.subcore_barrier()

    def edge_fn(srcbuf, dstbuf, ebuf, sgbuf, abuf):
        def edge(r, _):
            av = jnp.zeros((L,), jnp.float32)
            for v in range(8):
                sl = pl.ds(v * L, L)
                s2 = pl.ds(D + v * L, L)
                he = ebuf[r, sl] + srcbuf[r, sl] + dstbuf[r, sl]
                ebuf[r, sl] = he
                sgbuf[r, sl] = _sigmoid(he)
                av = av + srcbuf[r, s2] * dstbuf[r, s2]
            abuf[r, :] = _lanesum(av)
            return 0
        return edge

    NPAIR = NCH_A // 2

    def issue_src_idx(off0, off1):
        pltpu.async_copy(src_h.at[pl.ds(off0, CHA)], is0, m_i0)
        pltpu.async_copy(src_h.at[pl.ds(off1, CHA)], is1, m_i2)

    def issue_dst_idx(off0, off1):
        pltpu.async_copy(dst_h.at[pl.ds(off0, CHA)], id0, m_i1)
        pltpu.async_copy(dst_h.at[pl.ds(off1, CHA)], id1, m_i3)

    issue_src_idx(base, base + CHA)
    issue_dst_idx(base, base + CHA)

    def body(j, _):
        off0 = base + (2 * j) * CHA
        off1 = off0 + CHA
        pltpu.make_async_copy(src_h.at[pl.ds(off0, CHA)], is0, m_i0).wait()
        pltpu.make_async_copy(dst_h.at[pl.ds(off0, CHA)], id0, m_i1).wait()
        pltpu.make_async_copy(src_h.at[pl.ds(off1, CHA)], is1, m_i2).wait()
        pltpu.make_async_copy(dst_h.at[pl.ds(off1, CHA)], id1, m_i3).wait()
        gs0 = pltpu.async_copy(srcA_h.at[is0], src0, m_gs0)
        gd0 = pltpu.async_copy(dstA_h.at[id0], dst0, m_gd0)
        e0 = pltpu.async_copy(b3e_h.at[pl.ds(off0, CHA)], eb0, m_e0)
        gs1 = pltpu.async_copy(srcA_h.at[is1], src1, m_gs1)
        gd1 = pltpu.async_copy(dstA_h.at[id1], dst1, m_gd1)
        e1 = pltpu.async_copy(b3e_h.at[pl.ds(off1, CHA)], eb1, m_e1)
        gs0.wait()
        gd0.wait()
        e0.wait()
        lax.fori_loop(0, CHA, edge_fn(src0, dst0, eb0, sg0, ab0), 0)
        sc0 = pltpu.async_copy(sg0, acc.at[id0], m_sc0, add=True)
        wh0 = pltpu.async_copy(eb0, he_h.at[pl.ds(off0, CHA)], m_wh0)
        wa0 = pltpu.async_copy(ab0, al_h.at[pl.ds(off0, CHA)], m_wa0)
        gs1.wait()
        gd1.wait()
        e1.wait()

        @pl.when(j < NPAIR - 1)
        def _():
            issue_src_idx(off1 + CHA, off1 + 2 * CHA)

        lax.fori_loop(0, CHA, edge_fn(src1, dst1, eb1, sg1, ab1), 0)
        sc1 = pltpu.async_copy(sg1, acc.at[id1], m_sc1, add=True)
        wh1 = pltpu.async_copy(eb1, he_h.at[pl.ds(off1, CHA)], m_wh1)
        wa1 = pltpu.async_copy(ab1, al_h.at[pl.ds(off1, CHA)], m_wa1)
        sc0.wait()
        sc1.wait()

        @pl.when(j < NPAIR - 1)
        def _():
            issue_dst_idx(off1 + CHA, off1 + 2 * CHA)

        wh0.wait()
        wa0.wait()
        wh1.wait()
        wa1.wait()
        return 0

    lax.fori_loop(0, NPAIR, body, 0)

    off = base + (NCH_A - 1) * CHA
    pltpu.sync_copy(src_h.at[pl.ds(off, CHA)], is0)
    pltpu.sync_copy(dst_h.at[pl.ds(off, CHA)], id0)
    gs0 = pltpu.async_copy(srcA_h.at[is0], src0, m_gs0)
    gd0 = pltpu.async_copy(dstA_h.at[id0], dst0, m_gd0)
    pltpu.sync_copy(b3e_h.at[pl.ds(off, CHA)], eb0)
    gs0.wait()
    gd0.wait()
    lax.fori_loop(0, CHA, edge_fn(src0, dst0, eb0, sg0, ab0), 0)
    pltpu.sync_copy(sg0, acc.at[id0], add=True)
    pltpu.sync_copy(eb0, he_h.at[pl.ds(off, CHA)])
    pltpu.sync_copy(ab0, al_h.at[pl.ds(off, CHA)])
    plsc.subcore_barrier()
    for k in range(NZ):
        pltpu.sync_copy(acc.at[pl.ds(s * SLAB + k * ZR, ZR)], zbuf)
        pltpu.sync_copy(zbuf, ssig_h.at[c, pl.ds(s * SLAB + k * ZR, ZR)])

    @pl.when(s == 0)
    def _():
        pltpu.sync_copy(acc.at[pl.ds(NS * SLAB, TAIL)], zbuf.at[pl.ds(0, TAIL)])
        pltpu.sync_copy(zbuf.at[pl.ds(0, TAIL)],
                        ssig_h.at[c, pl.ds(NS * SLAB, TAIL)])


_pass_a = functools.partial(
    pl.kernel,
    out_type=[jax.ShapeDtypeStruct((E, D), jnp.float32),
              jax.ShapeDtypeStruct((E, L), jnp.float32),
              jax.ShapeDtypeStruct((NC, N, D), jnp.float32)],
    mesh=_mesh,
    scratch_types=([pltpu.VMEM((CHA,), jnp.int32),
                    pltpu.VMEM((CHA,), jnp.int32),
                    pltpu.VMEM((CHA, 2 * D), jnp.float32),
                    pltpu.VMEM((CHA, 2 * D), jnp.float32),
                    pltpu.VMEM((CHA, D), jnp.float32),
                    pltpu.VMEM((CHA, D), jnp.float32),
                    pltpu.VMEM((CHA, L), jnp.float32)] * 2
                   + [pltpu.VMEM((ZR, D), jnp.float32),
                      pltpu.VMEM_SHARED((N, D), jnp.float32)]
                   + [pltpu.SemaphoreType.DMA] * 16),
)(_pass_a_body)


def _pass_b_body(he_h, al_h, src_h, dst_h, vh_h, c2p_h,
                 acch_h, accp_h,
                 is0, id0, row0, he0, ab0, is1, id1, row1, he1, ab1,
                 zbuf, acc, sg0, sh0, sa0, ss0, sg1, sh1, sa1, ss1):
    c = lax.axis_index("c")
    s = lax.axis_index("s")
    base = s * EPS_B

    zv = jnp.zeros((L,), jnp.float32)

    def zb(i, _):
        r = i // 8
        col = (i % 8) * L
        zbuf[r, pl.ds(col, L)] = zv
        return 0

    lax.fori_loop(0, ZR * 8, zb, 0)
    for k in range(NZ):
        pltpu.sync_copy(zbuf, acc.at[pl.ds(s * SLAB + k * ZR, ZR)])

    @pl.when(s == 0)
    def _():
        pltpu.sync_copy(zbuf.at[pl.ds(0, TAIL)], acc.at[pl.ds(NS * SLAB, TAIL)])

    plsc.subcore_barrier()

    def make_loop(val_h, use_alpha):
        def edge_fn(rowbuf, hebuf, abuf):
            def edge(r, _):
                a = abuf[r]
                for v in range(8):
                    sl = pl.ds(v * L, L)
                    sg = _sigmoid(hebuf[r, sl])
                    if use_alpha:
                        rowbuf[r, sl] = sg * a * rowbuf[r, sl]
                    else:
                        rowbuf[r, sl] = sg * rowbuf[r, sl]
                return 0
            return edge

        def body(i, _):
            off0 = base + (2 * i) * CHB
            off1 = off0 + CHB
            pltpu.sync_copy(src_h.at[pl.ds(off0, CHB)], is0)
            pltpu.sync_copy(dst_h.at[pl.ds(off0, CHB)], id0)
            g0 = pltpu.async_copy(val_h.at[is0], row0, sg0)
            h0 = pltpu.async_copy(he_h.at[pl.ds(off0, CHB)], he0, sh0)
            if use_alpha:
                a0 = pltpu.async_copy(al_h.at[pl.ds(off0, CHB)], ab0, sa0)
            pltpu.sync_copy(src_h.at[pl.ds(off1, CHB)], is1)
            pltpu.sync_copy(dst_h.at[pl.ds(off1, CHB)], id1)
            g1 = pltpu.async_copy(val_h.at[is1], row1, sg1)
            h1 = pltpu.async_copy(he_h.at[pl.ds(off1, CHB)], he1, sh1)
            if use_alpha:
                a1 = pltpu.async_copy(al_h.at[pl.ds(off1, CHB)], ab1, sa1)
            g0.wait()
            h0.wait()
            if use_alpha:
                a0.wait()
            lax.fori_loop(0, CHB, edge_fn(row0, he0, ab0), 0)
            sc0 = pltpu.async_copy(row0, acc.at[id0], ss0, add=True)
            g1.wait()
            h1.wait()
            if use_alpha:
                a1.wait()
            lax.fori_loop(0, CHB, edge_fn(row1, he1, ab1), 0)
            sc1 = pltpu.async_copy(row1, acc.at[id1], ss1, add=True)
            sc0.wait()
            sc1.wait()
            return 0
        return body

    @pl.when(c == 0)
    def _():
        lax.fori_loop(0, NCH_B // 2, make_loop(vh_h, True), 0)

    @pl.when(c == 1)
    def _():
        lax.fori_loop(0, NCH_B // 2, make_loop(c2p_h, False), 0)

    plsc.subcore_barrier()
    for k in range(NZ):
        pltpu.sync_copy(acc.at[pl.ds(s * SLAB + k * ZR, ZR)], zbuf)

        @pl.when(c == 0)
        def _():
            pltpu.sync_copy(zbuf, acch_h.at[pl.ds(s * SLAB + k * ZR, ZR)])

        @pl.when(c == 1)
        def _():
            pltpu.sync_copy(zbuf, accp_h.at[pl.ds(s * SLAB + k * ZR, ZR)])

    @pl.when(s == 0)
    def _():
        pltpu.sync_copy(acc.at[pl.ds(NS * SLAB, TAIL)], zbuf.at[pl.ds(0, TAIL)])

        @pl.when(c == 0)
        def _():
            pltpu.sync_copy(zbuf.at[pl.ds(0, TAIL)],
                            acch_h.at[pl.ds(NS * SLAB, TAIL)])

        @pl.when(c == 1)
        def _():
            pltpu.sync_copy(zbuf.at[pl.ds(0, TAIL)],
                            accp_h.at[pl.ds(NS * SLAB, TAIL)])


_pass_b = functools.partial(
    pl.kernel,
    out_type=[jax.ShapeDtypeStruct((N, D), jnp.float32),
              jax.ShapeDtypeStruct((N, D), jnp.float32)],
    mesh=_mesh,
    scratch_types=[pltpu.VMEM((CHB,), jnp.int32),
                   pltpu.VMEM((CHB,), jnp.int32),
                   pltpu.VMEM((CHB, D), jnp.float32),
                   pltpu.VMEM((CHB, D), jnp.float32),
                   pltpu.VMEM((CHB, L), jnp.float32),
                   pltpu.VMEM((CHB,), jnp.int32),
                   pltpu.VMEM((CHB,), jnp.int32),
                   pltpu.VMEM((CHB, D), jnp.float32),
                   pltpu.VMEM((CHB, D), jnp.float32),
                   pltpu.VMEM((CHB, L), jnp.float32),
                   pltpu.VMEM((ZR, D), jnp.float32),
                   pltpu.VMEM_SHARED((N, D), jnp.float32),
                   pltpu.SemaphoreType.DMA,
                   pltpu.SemaphoreType.DMA,
                   pltpu.SemaphoreType.DMA,
                   pltpu.SemaphoreType.DMA,
                   pltpu.SemaphoreType.DMA,
                   pltpu.SemaphoreType.DMA,
                   pltpu.SemaphoreType.DMA,
                   pltpu.SemaphoreType.DMA],
)(_pass_b_body)


def kernel(h, e, p, edge_src, edge_dst, VW, Vb, KW, Kb, B1W, B1b, B2W, B2b,
           B3W, B3b, C1W, C1b, C2W, C2b, bnh_g, bnh_b, bne_g, bne_b, ln_g,
           ln_b):
    vh, c1p, c2p, srcA, dstA = _node_dense(
        h, p, VW, Vb, KW, Kb, B1W, B1b, B2W, B2b, C1W, C1b, C2W, C2b)
    b3e = _edge_dense(e, B3W, B3b)

    hat_eta, alpha, ssig_parts = _pass_a(
        b3e, edge_src, edge_dst, srcA, dstA)
    stats = _estats(hat_eta)
    acc_h, acc_p = _pass_b(hat_eta, alpha, edge_src, edge_dst, vh, c2p)

    h_new, p_new, em, ei = _finalize(
        vh, c1p, acc_h, acc_p, ssig_parts, stats, bnh_g, bnh_b, ln_g, ln_b)
    e_new = _enew(hat_eta, em, ei, bne_g, bne_b)
    return (h_new, e_new, p_new)


# speedup vs baseline: 1.7430x; 1.7430x over previous
import functools

import jax
import jax.numpy as jnp
from jax import lax
from jax.experimental import pallas as pl
from jax.experimental.pallas import tpu as pltpu
from jax.experimental.pallas import tpu_sc as plsc

N = 10000
E = 320000
D = 128
EPS = 1e-12

NC = 2
NS = 16
L = 16
NW = NC * NS

CHA = 16
EPW_A = E // NW
NCH_A = EPW_A // CHA
CHB = 40
EPS_B = E // NS
NCH_B = EPS_B // CHB
SLAB = 624
ZR = 16
NZ = SLAB // ZR
TAIL = N - NS * SLAB

_mesh = plsc.VectorSubcoreMesh(core_axis_name="c", subcore_axis_name="s")


def _sigmoid(x):
    return 1.0 / (1.0 + jnp.exp(-x))


def _lanesum(v):
    lanes = lax.iota(jnp.int32, L)
    for sh in (1, 2, 4, 8):
        v = v + jnp.take(v, lanes ^ sh, axis=0)
    return v


def _pack_bf16(x):
    m = x.shape[0]
    xb = x.astype(jnp.bfloat16).reshape(m, 2 * D // 32, 2, L)
    u = lax.bitcast_convert_type(xb, jnp.uint16).astype(jnp.uint32)
    return (u[:, :, 0, :] | (u[:, :, 1, :] << 16)).reshape(m, D)


def _unpack(u):
    lo = lax.bitcast_convert_type(u << 16, jnp.float32)
    hi = lax.bitcast_convert_type(u & jnp.uint32(0xFFFF0000), jnp.float32)
    return lo, hi


def _node_dense_body(h_ref, p_ref, VW1, VW2, Vb, KW1, KW2, Kb, B1W, B1b, B2W,
                     B2b, C1W, C1b, C2W, C2b, vh_o, c1_o, c2_o, srcA_o,
                     dstA_o):
    h = h_ref[...]
    p = p_ref[...]
    vh_o[...] = h @ VW1[...] + p @ VW2[...] + Vb[...]
    qh = h @ KW1[...] + p @ KW2[...] + Kb[...]
    srcA_o[:, :D] = h @ B1W[...] + B1b[...]
    srcA_o[:, D:] = jnp.exp(jnp.tanh(qh))
    dstA_o[:, :D] = h @ B2W[...] + B2b[...]
    dstA_o[:, D:] = jnp.exp(_sigmoid(qh))
    c1_o[...] = p @ C1W[...] + C1b[...]
    c2_o[...] = p @ C2W[...] + C2b[...]


def _node_dense(h, p, VW, Vb, KW, Kb, B1W, B1b, B2W, B2b, C1W, C1b, C2W, C2b):
    R = 2000
    grid = N // R
    row = pl.BlockSpec((R, D), lambda i: (i, 0))
    row2 = pl.BlockSpec((R, 2 * D), lambda i: (i, 0))
    full = pl.BlockSpec((D, D), lambda i: (0, 0))
    vec = pl.BlockSpec((D,), lambda i: (0,))
    return pl.pallas_call(
        _node_dense_body,
        grid=grid,
        in_specs=[row, row, full, full, vec, full, full, vec,
                  full, vec, full, vec, full, vec, full, vec],
        out_specs=[row, row, row, row2, row2],
        out_shape=[jax.ShapeDtypeStruct((N, D), jnp.float32),
                   jax.ShapeDtypeStruct((N, D), jnp.float32),
                   jax.ShapeDtypeStruct((N, D), jnp.float32),
                   jax.ShapeDtypeStruct((N, 2 * D), jnp.float32),
                   jax.ShapeDtypeStruct((N, 2 * D), jnp.float32)],
    )(h, p, VW[:D], VW[D:], Vb, KW[:D], KW[D:], Kb, B1W, B1b, B2W, B2b,
      C1W, C1b, C2W, C2b)


def _edge_dense_body(e_ref, W, b, o_ref):
    o_ref[...] = e_ref[...] @ W[...] + b[...]


def _edge_dense(e, B3W, B3b):
    R = 2000
    return pl.pallas_call(
        _edge_dense_body,
        grid=E // R,
        in_specs=[pl.BlockSpec((R, D), lambda i: (i, 0)),
                  pl.BlockSpec((D, D), lambda i: (0, 0)),
                  pl.BlockSpec((D,), lambda i: (0,))],
        out_specs=pl.BlockSpec((R, D), lambda i: (i, 0)),
        out_shape=jax.ShapeDtypeStruct((E, D), jnp.float32),
    )(e, B3W, B3b)


def _estats_body(he_ref, st_o):
    i = pl.program_id(0)

    @pl.when(i == 0)
    def _():
        st_o[...] = jnp.zeros_like(st_o)

    he = he_ref[...]
    st_o[0, :] += jnp.sum(he, axis=0)
    st_o[1, :] += jnp.sum(he * he, axis=0)


def _estats(hat_eta):
    R = 4000
    return pl.pallas_call(
        _estats_body,
        grid=E // R,
        in_specs=[pl.BlockSpec((R, D), lambda i: (i, 0))],
        out_specs=pl.BlockSpec((2, D), lambda i: (0, 0)),
        out_shape=jax.ShapeDtypeStruct((2, D), jnp.float32),
    )(hat_eta)


def _finalize_body(vh_ref, c1_ref, acch_ref, accp_ref, ssig_ref, st_ref,
                   bnh_g, bnh_b, ln_g, ln_b, h_o, p_o, em_o, ei_o):
    den = ssig_ref[0] + ssig_ref[1] + EPS
    h = vh_ref[...] + acch_ref[...] / den
    m = jnp.mean(h, axis=0, keepdims=True)
    v = jnp.mean((h - m) ** 2, axis=0, keepdims=True)
    h = (h - m) * lax.rsqrt(v + 1e-5) * bnh_g[...] + bnh_b[...]
    h = jnp.maximum(h, 0.0)
    lm = jnp.mean(h, axis=-1, keepdims=True)
    lv = jnp.mean((h - lm) ** 2, axis=-1, keepdims=True)
    h_o[...] = (h - lm) * lax.rsqrt(lv + 1e-5) * ln_g[...] + ln_b[...]
    p_o[...] = jnp.tanh(c1_ref[...] + accp_ref[...] / den)
    em = st_ref[0] / E
    ev = st_ref[1] / E - em * em
    em_o[...] = em.reshape(1, D)
    ei_o[...] = lax.rsqrt(ev + 1e-5).reshape(1, D)


def _finalize(vh, c1p, acc_h, acc_p, ssig_parts, stats, bnh_g, bnh_b, ln_g,
              ln_b):
    nodes = pl.BlockSpec((N, D), lambda: (0, 0))
    vec = pl.BlockSpec((D,), lambda: (0,))
    return pl.pallas_call(
        _finalize_body,
        in_specs=[nodes, nodes, nodes, nodes,
                  pl.BlockSpec((NC, N, D), lambda: (0, 0, 0)),
                  pl.BlockSpec((2, D), lambda: (0, 0)),
                  vec, vec, vec, vec],
        out_specs=[nodes, nodes, pl.BlockSpec((1, D), lambda: (0, 0)),
                   pl.BlockSpec((1, D), lambda: (0, 0))],
        out_shape=[jax.ShapeDtypeStruct((N, D), jnp.float32),
                   jax.ShapeDtypeStruct((N, D), jnp.float32),
                   jax.ShapeDtypeStruct((1, D), jnp.float32),
                   jax.ShapeDtypeStruct((1, D), jnp.float32)],
    )(vh, c1p, acc_h, acc_p, ssig_parts, stats, bnh_g, bnh_b, ln_g, ln_b)


def _enew_body(he_ref, em_ref, ei_ref, g, b, o_ref):
    x = (he_ref[...] - em_ref[...]) * ei_ref[...] * g[...] + b[...]
    o_ref[...] = jnp.maximum(x, 0.0)


def _enew(hat_eta, em, ei, bne_g, bne_b):
    R = 2000
    return pl.pallas_call(
        _enew_body,
        grid=E // R,
        in_specs=[pl.BlockSpec((R, D), lambda i: (i, 0)),
                  pl.BlockSpec((1, D), lambda i: (0, 0)),
                  pl.BlockSpec((1, D), lambda i: (0, 0)),
                  pl.BlockSpec((D,), lambda i: (0,)),
                  pl.BlockSpec((D,), lambda i: (0,))],
        out_specs=pl.BlockSpec((R, D), lambda i: (i, 0)),
        out_shape=jax.ShapeDtypeStruct((E, D), jnp.float32),
    )(hat_eta, em, ei, bne_g, bne_b)


def _pass_a_body(b3e_h, src_h, dst_h, srcA_h, dstA_h,
                 he_h, al_h, ssig_h,
                 is0, id0, src0, dst0, eb0, sg0, ab0,
                 is1, id1, src1, dst1, eb1, sg1, ab1,
                 zbuf, acc,
                 m_gs0, m_gd0, m_e0, m_sc0, m_wh0, m_wa0,
                 m_gs1, m_gd1, m_e1, m_sc1, m_wh1, m_wa1,
                 m_i0, m_i1, m_i2, m_i3):
    c = lax.axis_index("c")
    s = lax.axis_index("s")
    wid = s * NC + c
    base = wid * EPW_A

    zv = jnp.zeros((L,), jnp.float32)

    def zb(i, _):
        r = i // 8
        col = (i % 8) * L
        zbuf[r, pl.ds(col, L)] = zv
        return 0

    lax.fori_loop(0, ZR * 8, zb, 0)
    for k in range(NZ):
        pltpu.sync_copy(zbuf, acc.at[pl.ds(s * SLAB + k * ZR, ZR)])

    @pl.when(s == 0)
    def _():
        pltpu.sync_copy(zbuf.at[pl.ds(0, TAIL)], acc.at[pl.ds(NS * SLAB, TAIL)])

    plsc.subcore_barrier()

    def edge_fn(srcbuf, dstbuf, ebuf, sgbuf, abuf):
        def edge(r, _):
            av = jnp.zeros((L,), jnp.float32)
            for w in range(4):
                slo, shi = _unpack(srcbuf[r, pl.ds(w * L, L)])
                dlo, dhi = _unpack(dstbuf[r, pl.ds(w * L, L)])
                sl0 = pl.ds(2 * w * L, L)
                sl1 = pl.ds((2 * w + 1) * L, L)
                he0 = ebuf[r, sl0] + slo + dlo
                ebuf[r, sl0] = he0
                sgbuf[r, sl0] = _sigmoid(he0)
                he1 = ebuf[r, sl1] + shi + dhi
                ebuf[r, sl1] = he1
                sgbuf[r, sl1] = _sigmoid(he1)
            for w in range(4, 8):
                slo, shi = _unpack(srcbuf[r, pl.ds(w * L, L)])
                dlo, dhi = _unpack(dstbuf[r, pl.ds(w * L, L)])
                av = av + slo * dlo + shi * dhi
            abuf[r, :] = _lanesum(av)
            return 0
        return edge

    NPAIR = NCH_A // 2

    def issue_src_idx(off0, off1):
        pltpu.async_copy(src_h.at[pl.ds(off0, CHA)], is0, m_i0)
        pltpu.async_copy(src_h.at[pl.ds(off1, CHA)], is1, m_i2)

    def issue_dst_idx(off0, off1):
        pltpu.async_copy(dst_h.at[pl.ds(off0, CHA)], id0, m_i1)
        pltpu.async_copy(dst_h.at[pl.ds(off1, CHA)], id1, m_i3)

    issue_src_idx(base, base + CHA)
    issue_dst_idx(base, base + CHA)

    def body(j, _):
        off0 = base + (2 * j) * CHA
        off1 = off0 + CHA
        pltpu.make_async_copy(src_h.at[pl.ds(off0, CHA)], is0, m_i0).wait()
        pltpu.make_async_copy(dst_h.at[pl.ds(off0, CHA)], id0, m_i1).wait()
        pltpu.make_async_copy(src_h.at[pl.ds(off1, CHA)], is1, m_i2).wait()
        pltpu.make_async_copy(dst_h.at[pl.ds(off1, CHA)], id1, m_i3).wait()
        gs0 = pltpu.async_copy(srcA_h.at[is0], src0, m_gs0)
        gd0 = pltpu.async_copy(dstA_h.at[id0], dst0, m_gd0)
        e0 = pltpu.async_copy(b3e_h.at[pl.ds(off0, CHA)], eb0, m_e0)
        gs1 = pltpu.async_copy(srcA_h.at[is1], src1, m_gs1)
        gd1 = pltpu.async_copy(dstA_h.at[id1], dst1, m_gd1)
        e1 = pltpu.async_copy(b3e_h.at[pl.ds(off1, CHA)], eb1, m_e1)
        gs0.wait()
        gd0.wait()
        e0.wait()
        lax.fori_loop(0, CHA, edge_fn(src0, dst0, eb0, sg0, ab0), 0)
        sc0 = pltpu.async_copy(sg0, acc.at[id0], m_sc0, add=True)
        wh0 = pltpu.async_copy(eb0, he_h.at[pl.ds(off0, CHA)], m_wh0)
        wa0 = pltpu.async_copy(ab0, al_h.at[pl.ds(off0, CHA)], m_wa0)
        gs1.wait()
        gd1.wait()
        e1.wait()

        @pl.when(j < NPAIR - 1)
        def _():
            issue_src_idx(off1 + CHA, off1 + 2 * CHA)

        lax.fori_loop(0, CHA, edge_fn(src1, dst1, eb1, sg1, ab1), 0)
        sc1 = pltpu.async_copy(sg1, acc.at[id1], m_sc1, add=True)
        wh1 = pltpu.async_copy(eb1, he_h.at[pl.ds(off1, CHA)], m_wh1)
        wa1 = pltpu.async_copy(ab1, al_h.at[pl.ds(off1, CHA)], m_wa1)
        sc0.wait()
        sc1.wait()

        @pl.when(j < NPAIR - 1)
        def _():
            issue_dst_idx(off1 + CHA, off1 + 2 * CHA)

        wh0.wait()
        wa0.wait()
        wh1.wait()
        wa1.wait()
        return 0

    lax.fori_loop(0, NPAIR, body, 0)

    off = base + (NCH_A - 1) * CHA
    pltpu.sync_copy(src_h.at[pl.ds(off, CHA)], is0)
    pltpu.sync_copy(dst_h.at[pl.ds(off, CHA)], id0)
    gs0 = pltpu.async_copy(srcA_h.at[is0], src0, m_gs0)
    gd0 = pltpu.async_copy(dstA_h.at[id0], dst0, m_gd0)
    pltpu.sync_copy(b3e_h.at[pl.ds(off, CHA)], eb0)
    gs0.wait()
    gd0.wait()
    lax.fori_loop(0, CHA, edge_fn(src0, dst0, eb0, sg0, ab0), 0)
    pltpu.sync_copy(sg0, acc.at[id0], add=True)
    pltpu.sync_copy(eb0, he_h.at[pl.ds(off, CHA)])
    pltpu.sync_copy(ab0, al_h.at[pl.ds(off, CHA)])
    plsc.subcore_barrier()
    for k in range(NZ):
        pltpu.sync_copy(acc.at[pl.ds(s * SLAB + k * ZR, ZR)], zbuf)
        pltpu.sync_copy(zbuf, ssig_h.at[c, pl.ds(s * SLAB + k * ZR, ZR)])

    @pl.when(s == 0)
    def _():
        pltpu.sync_copy(acc.at[pl.ds(NS * SLAB, TAIL)], zbuf.at[pl.ds(0, TAIL)])
        pltpu.sync_copy(zbuf.at[pl.ds(0, TAIL)],
                        ssig_h.at[c, pl.ds(NS * SLAB, TAIL)])


_pass_a = functools.partial(
    pl.kernel,
    out_type=[jax.ShapeDtypeStruct((E, D), jnp.float32),
              jax.ShapeDtypeStruct((E, L), jnp.float32),
              jax.ShapeDtypeStruct((NC, N, D), jnp.float32)],
    mesh=_mesh,
    scratch_types=([pltpu.VMEM((CHA,), jnp.int32),
                    pltpu.VMEM((CHA,), jnp.int32),
                    pltpu.VMEM((CHA, D), jnp.uint32),
                    pltpu.VMEM((CHA, D), jnp.uint32),
                    pltpu.VMEM((CHA, D), jnp.float32),
                    pltpu.VMEM((CHA, D), jnp.float32),
                    pltpu.VMEM((CHA, L), jnp.float32)] * 2
                   + [pltpu.VMEM((ZR, D), jnp.float32),
                      pltpu.VMEM_SHARED((N, D), jnp.float32)]
                   + [pltpu.SemaphoreType.DMA] * 16),
)(_pass_a_body)


def _pass_b_body(he_h, al_h, src_h, dst_h, vh_h, c2p_h,
                 acch_h, accp_h,
                 is0, id0, row0, he0, ab0, is1, id1, row1, he1, ab1,
                 zbuf, acc, sg0, sh0, sa0, ss0, sg1, sh1, sa1, ss1):
    c = lax.axis_index("c")
    s = lax.axis_index("s")
    base = s * EPS_B

    zv = jnp.zeros((L,), jnp.float32)

    def zb(i, _):
        r = i // 8
        col = (i % 8) * L
        zbuf[r, pl.ds(col, L)] = zv
        return 0

    lax.fori_loop(0, ZR * 8, zb, 0)
    for k in range(NZ):
        pltpu.sync_copy(zbuf, acc.at[pl.ds(s * SLAB + k * ZR, ZR)])

    @pl.when(s == 0)
    def _():
        pltpu.sync_copy(zbuf.at[pl.ds(0, TAIL)], acc.at[pl.ds(NS * SLAB, TAIL)])

    plsc.subcore_barrier()

    def make_loop(val_h, use_alpha):
        def edge_fn(rowbuf, hebuf, abuf):
            def edge(r, _):
                a = abuf[r]
                for v in range(8):
                    sl = pl.ds(v * L, L)
                    sg = _sigmoid(hebuf[r, sl])
                    if use_alpha:
                        rowbuf[r, sl] = sg * a * rowbuf[r, sl]
                    else:
                        rowbuf[r, sl] = sg * rowbuf[r, sl]
                return 0
            return edge

        def body(i, _):
            off0 = base + (2 * i) * CHB
            off1 = off0 + CHB
            pltpu.sync_copy(src_h.at[pl.ds(off0, CHB)], is0)
            pltpu.sync_copy(dst_h.at[pl.ds(off0, CHB)], id0)
            g0 = pltpu.async_copy(val_h.at[is0], row0, sg0)
            h0 = pltpu.async_copy(he_h.at[pl.ds(off0, CHB)], he0, sh0)
            if use_alpha:
                a0 = pltpu.async_copy(al_h.at[pl.ds(off0, CHB)], ab0, sa0)
            pltpu.sync_copy(src_h.at[pl.ds(off1, CHB)], is1)
            pltpu.sync_copy(dst_h.at[pl.ds(off1, CHB)], id1)
            g1 = pltpu.async_copy(val_h.at[is1], row1, sg1)
            h1 = pltpu.async_copy(he_h.at[pl.ds(off1, CHB)], he1, sh1)
            if use_alpha:
                a1 = pltpu.async_copy(al_h.at[pl.ds(off1, CHB)], ab1, sa1)
            g0.wait()
            h0.wait()
            if use_alpha:
                a0.wait()
            lax.fori_loop(0, CHB, edge_fn(row0, he0, ab0), 0)
            sc0 = pltpu.async_copy(row0, acc.at[id0], ss0, add=True)
            g1.wait()
            h1.wait()
            if use_alpha:
                a1.wait()
            lax.fori_loop(0, CHB, edge_fn(row1, he1, ab1), 0)
            sc1 = pltpu.async_copy(row1, acc.at[id1], ss1, add=True)
            sc0.wait()
            sc1.wait()
            return 0
        return body

    @pl.when(c == 0)
    def _():
        lax.fori_loop(0, NCH_B // 2, make_loop(vh_h, True), 0)

    @pl.when(c == 1)
    def _():
        lax.fori_loop(0, NCH_B // 2, make_loop(c2p_h, False), 0)

    plsc.subcore_barrier()
    for k in range(NZ):
        pltpu.sync_copy(acc.at[pl.ds(s * SLAB + k * ZR, ZR)], zbuf)

        @pl.when(c == 0)
        def _():
            pltpu.sync_copy(zbuf, acch_h.at[pl.ds(s * SLAB + k * ZR, ZR)])

        @pl.when(c == 1)
        def _():
            pltpu.sync_copy(zbuf, accp_h.at[pl.ds(s * SLAB + k * ZR, ZR)])

    @pl.when(s == 0)
    def _():
        pltpu.sync_copy(acc.at[pl.ds(NS * SLAB, TAIL)], zbuf.at[pl.ds(0, TAIL)])

        @pl.when(c == 0)
        def _():
            pltpu.sync_copy(zbuf.at[pl.ds(0, TAIL)],
                            acch_h.at[pl.ds(NS * SLAB, TAIL)])

        @pl.when(c == 1)
        def _():
            pltpu.sync_copy(zbuf.at[pl.ds(0, TAIL)],
                            accp_h.at[pl.ds(NS * SLAB, TAIL)])


_pass_b = functools.partial(
    pl.kernel,
    out_type=[jax.ShapeDtypeStruct((N, D), jnp.float32),
              jax.ShapeDtypeStruct((N, D), jnp.float32)],
    mesh=_mesh,
    scratch_types=[pltpu.VMEM((CHB,), jnp.int32),
                   pltpu.VMEM((CHB,), jnp.int32),
                   pltpu.VMEM((CHB, D), jnp.float32),
                   pltpu.VMEM((CHB, D), jnp.float32),
                   pltpu.VMEM((CHB, L), jnp.float32),
                   pltpu.VMEM((CHB,), jnp.int32),
                   pltpu.VMEM((CHB,), jnp.int32),
                   pltpu.VMEM((CHB, D), jnp.float32),
                   pltpu.VMEM((CHB, D), jnp.float32),
                   pltpu.VMEM((CHB, L), jnp.float32),
                   pltpu.VMEM((ZR, D), jnp.float32),
                   pltpu.VMEM_SHARED((N, D), jnp.float32),
                   pltpu.SemaphoreType.DMA,
                   pltpu.SemaphoreType.DMA,
                   pltpu.SemaphoreType.DMA,
                   pltpu.SemaphoreType.DMA,
                   pltpu.SemaphoreType.DMA,
                   pltpu.SemaphoreType.DMA,
                   pltpu.SemaphoreType.DMA,
                   pltpu.SemaphoreType.DMA],
)(_pass_b_body)


def kernel(h, e, p, edge_src, edge_dst, VW, Vb, KW, Kb, B1W, B1b, B2W, B2b,
           B3W, B3b, C1W, C1b, C2W, C2b, bnh_g, bnh_b, bne_g, bne_b, ln_g,
           ln_b):
    vh, c1p, c2p, srcA, dstA = _node_dense(
        h, p, VW, Vb, KW, Kb, B1W, B1b, B2W, B2b, C1W, C1b, C2W, C2b)
    b3e = _edge_dense(e, B3W, B3b)

    hat_eta, alpha, ssig_parts = _pass_a(
        b3e, edge_src, edge_dst, _pack_bf16(srcA), _pack_bf16(dstA))
    stats = _estats(hat_eta)
    acc_h, acc_p = _pass_b(hat_eta, alpha, edge_src, edge_dst, vh, c2p)

    h_new, p_new, em, ei = _finalize(
        vh, c1p, acc_h, acc_p, ssig_parts, stats, bnh_g, bnh_b, ln_g, ln_b)
    e_new = _enew(hat_eta, em, ei, bne_g, bne_b)
    return (h_new, e_new, p_new)
